# baseline (speedup 1.0000x reference)
import functools

import jax
import jax.numpy as jnp
from jax.experimental import pallas as pl
from jax.experimental.pallas import tpu as pltpu

N_USERS = 50000
N_ITEMS = 50000
N_NODES = N_USERS + N_ITEMS
D = 64
N_LAYERS = 3
POP_BINS = 10
B = 1024

ITEM_BLK = 1024


def _score_block_kernel(usel_ref, ubias_ref, items_ref, pops_ref, ibias_ref,
                        uw1_ref, ub1_ref, uw2_ref, ub2_ref,
                        iw1_ref, ib1_ref, iw2_ref, ib2_ref,
                        pop_emb_ref, gw1_ref, gb1_ref, gw2_ref, gb2_ref,
                        out_ref):
    usel = usel_ref[...]
    uh = jnp.maximum(
        jnp.dot(usel, uw1_ref[...], preferred_element_type=jnp.float32)
        + ub1_ref[...][None, :], 0.0)
    u = (jnp.dot(uh, uw2_ref[...], preferred_element_type=jnp.float32)
         + ub2_ref[...][None, :])

    items = items_ref[...]
    pops = pops_ref[...]
    onehot = (pops[:, None] == jax.lax.broadcasted_iota(
        jnp.int32, (1, POP_BINS), 1)).astype(jnp.float32)
    pvec = jnp.dot(onehot, pop_emb_ref[...], preferred_element_type=jnp.float32)

    gate_in = jnp.concatenate([items, pvec], axis=-1)
    h = jnp.maximum(
        jnp.dot(gate_in, gw1_ref[...], preferred_element_type=jnp.float32)
        + gb1_ref[...][None, :], 0.0)
    zlog = (jnp.dot(h, gw2_ref[...], preferred_element_type=jnp.float32)
            + gb2_ref[...][None, :])
    z = jax.nn.sigmoid(zlog)
    fused = (1.0 - z) * items + z * pvec

    ih = jnp.maximum(
        jnp.dot(fused, iw1_ref[...], preferred_element_type=jnp.float32)
        + ib1_ref[...][None, :], 0.0)
    it = (jnp.dot(ih, iw2_ref[...], preferred_element_type=jnp.float32)
          + ib2_ref[...][None, :])

    scores = jax.lax.dot_general(
        u, it, (((1,), (1,)), ((), ())), preferred_element_type=jnp.float32)
    out_ref[...] = scores + ubias_ref[...][:, None] + ibias_ref[...][None, :]


def _score_pallas(usel, ubias, all_items, item_pop_bins, ibias,
                  uw1, ub1, uw2, ub2, iw1, ib1, iw2, ib2,
                  pop_emb, gw1, gb1, gw2, gb2):
    n_blocks = pl.cdiv(N_ITEMS, ITEM_BLK)
    full = lambda shape: pl.BlockSpec(shape, lambda j: tuple(0 for _ in shape))
    grid_spec = pl.GridSpec(
        grid=(n_blocks,),
        in_specs=[
            full((B, D)),
            full((B,)),
            pl.BlockSpec((ITEM_BLK, D), lambda j: (j, 0)),
            pl.BlockSpec((ITEM_BLK,), lambda j: (j,)),
            pl.BlockSpec((ITEM_BLK,), lambda j: (j,)),
            full((D, D)), full((D,)), full((D, D)), full((D,)),
            full((D, D)), full((D,)), full((D, D)), full((D,)),
            full((POP_BINS, D)),
            full((2 * D, D)), full((D,)), full((D, 1)), full((1,)),
        ],
        out_specs=pl.BlockSpec((B, ITEM_BLK), lambda j: (0, j)),
    )
    return pl.pallas_call(
        _score_block_kernel,
        grid_spec=grid_spec,
        out_shape=jax.ShapeDtypeStruct((B, N_ITEMS), jnp.float32),
    )(usel, ubias, all_items, item_pop_bins, ibias,
      uw1, ub1, uw2, ub2, iw1, ib1, iw2, ib2,
      pop_emb, gw1, gb1, gw2, gb2)


def kernel(users, edge_index, item_pop_bins, user_emb, item_emb, user_bias,
           item_bias, uw1, ub1, uw2, ub2, iw1, ib1, iw2, ib2,
           pop_emb, gw1, gb1, gw2, gb2):
    u = edge_index[0]
    it = edge_index[1] + N_USERS
    src = jnp.concatenate([u, it])
    dst = jnp.concatenate([it, u])
    deg = jnp.zeros((N_NODES,), jnp.float32).at[src].add(1.0)
    deg = jnp.maximum(deg, 1.0)
    norm = 1.0 / jnp.sqrt(deg[src] * deg[dst])
    all_emb = jnp.concatenate([user_emb, item_emb], axis=0)
    embs = [all_emb]
    for _ in range(N_LAYERS):
        msg = embs[-1][src] * norm[:, None]
        nxt = jax.ops.segment_sum(msg, dst, num_segments=N_NODES)
        embs.append(nxt)
    stack = jnp.stack(embs, axis=0)
    out = stack.mean(axis=0)
    all_users, all_items = out[:N_USERS], out[N_USERS:]

    usel = all_users[users]
    ubias = user_bias[users].reshape(-1)
    ibias = item_bias.reshape(-1)
    return _score_pallas(usel, ubias, all_items, item_pop_bins, ibias,
                         uw1, ub1, uw2, ub2, iw1, ib1, iw2, ib2,
                         pop_emb, gw1, gb1, gw2, gb2)
